# Initial kernel scaffold; baseline (speedup 1.0000x reference)
import jax
import jax.numpy as jnp
from jax.experimental import pallas as pl


def kernel(x, w):
    raise NotImplementedError("write your pallas kernel here")


# speedup vs baseline: 11.6606x; 11.6606x over previous
import jax
import jax.numpy as jnp
from jax.experimental import pallas as pl

_C = 32
_T = 4
_F = 128
_L = 512
_N = _F * _L
_K = 128
_D = _C * _T
_INT_MIN = -2147483648


def _attn_kernel(x_ref, w_ref, out_ref):
    X = x_ref[0]
    w = w_ref[0, 0]

    colsum = jnp.sum(X, axis=1)
    total = jnp.sum(colsum, axis=1, keepdims=True)
    last = colsum[:, _L - 1:_L]
    qs = []
    prefix = jnp.zeros_like(total)
    for i in range(_T):
        qs.append((total - prefix + i * last) * (1.0 / _N))
        if i < _T - 1:
            prefix = prefix + colsum[:, i:i + 1]
    Qt = jnp.concatenate(qs, axis=1)

    Xm = X.reshape(_C, _N)
    s = jax.lax.dot_general(Qt, Xm, (((0,), (0,)), ((), ())),
                            preferred_element_type=jnp.float32)
    s3 = s.reshape(_T, _F, _L)

    score = s3[0]
    for i in range(1, _T):
        body = s3[i, :, i:]
        tail = jnp.broadcast_to(s3[i, :, _L - 1:_L], (_F, i))
        score = score + jnp.concatenate([body, tail], axis=1)

    bits = jax.lax.bitcast_convert_type(score, jnp.int32)
    key = jnp.where(bits >= 0, bits, jnp.int32(_INT_MIN) - bits)

    def cnt_ge(t):
        return jnp.sum((key >= t).astype(jnp.int32))

    t0 = jnp.where(cnt_ge(jnp.int32(0)) >= _K, jnp.int32(0),
                   jnp.int32(_INT_MIN))

    def t_body(k, t):
        cand = t | jax.lax.shift_left(jnp.int32(1), jnp.int32(30) - k)
        return jnp.where(cnt_ge(cand) >= _K, cand, t)

    tstar = jax.lax.fori_loop(0, 31, t_body, t0, unroll=True)

    gt = key > tstar
    eq = key == tstar
    need = _K - jnp.sum(gt.astype(jnp.int32))
    idx = (jax.lax.broadcasted_iota(jnp.int32, (_F, _L), 0) * _L
           + jax.lax.broadcasted_iota(jnp.int32, (_F, _L), 1))

    def j_body(k, jm):
        cand = jm | jax.lax.shift_left(jnp.int32(1), jnp.int32(16) - k)
        c = jnp.sum((eq & (idx < cand)).astype(jnp.int32))
        return jnp.where(c < need, cand, jm)

    jm = jax.lax.fori_loop(0, 17, j_body, jnp.int32(0), unroll=True)
    mask = gt | (eq & (idx <= jm))

    masked = jnp.where(mask, score, -jnp.inf)
    m = jnp.max(masked)
    e = jnp.where(mask, jnp.exp(score - m), 0.0)
    Z = jnp.sum(e)

    As = []
    for i in range(_T):
        if i == 0:
            As.append(e)
        else:
            zeros = jnp.zeros((_F, i), jnp.float32)
            bodyp = e[:, :_L - 1 - i]
            tailp = jnp.sum(e[:, _L - 1 - i:], axis=1, keepdims=True)
            As.append(jnp.concatenate([zeros, bodyp, tailp], axis=1))
    A = jnp.stack(As, axis=0).reshape(_T, _N)

    outdot = jax.lax.dot_general(Xm, A, (((1,), (1,)), ((), ())),
                                 preferred_element_type=jnp.float32)
    G = (w / Z) * outdot + (0.5 - w) * Qt
    out_ref[0] = G.T


def kernel(x, w):
    B = x.shape[0]
    w2 = jnp.asarray(w, jnp.float32).reshape(1, 1)
    out = pl.pallas_call(
        _attn_kernel,
        grid=(B,),
        in_specs=[
            pl.BlockSpec((1, _C, _F, _L), lambda b: (b, 0, 0, 0)),
            pl.BlockSpec((1, 1), lambda b: (0, 0)),
        ],
        out_specs=pl.BlockSpec((1, _T, _C), lambda b: (b, 0, 0)),
        out_shape=jax.ShapeDtypeStruct((B, _T, _C), jnp.float32),
    )(x, w2)
    return out.reshape(B, _C, 1, _T)


# speedup vs baseline: 11.6646x; 1.0003x over previous
import jax
import jax.numpy as jnp
from jax.experimental import pallas as pl
from jax.experimental.pallas import tpu as pltpu

_C = 32
_T = 4
_F = 128
_L = 512
_N = _F * _L
_K = 128
_D = _C * _T
_INT_MIN = -2147483648


def _attn_kernel(x_ref, w_ref, out_ref):
    X = x_ref[0]
    w = w_ref[0, 0]

    colsum = jnp.sum(X, axis=1)
    total = jnp.sum(colsum, axis=1, keepdims=True)
    last = colsum[:, _L - 1:_L]
    qs = []
    prefix = jnp.zeros_like(total)
    for i in range(_T):
        qs.append((total - prefix + i * last) * (1.0 / _N))
        if i < _T - 1:
            prefix = prefix + colsum[:, i:i + 1]
    Qt = jnp.concatenate(qs, axis=1)

    Xm = X.reshape(_C, _N)
    s = jax.lax.dot_general(Qt, Xm, (((0,), (0,)), ((), ())),
                            preferred_element_type=jnp.float32)
    s3 = s.reshape(_T, _F, _L)

    score = s3[0]
    for i in range(1, _T):
        body = s3[i, :, i:]
        tail = jnp.broadcast_to(s3[i, :, _L - 1:_L], (_F, i))
        score = score + jnp.concatenate([body, tail], axis=1)

    bits = jax.lax.bitcast_convert_type(score, jnp.int32)
    key = jnp.where(bits >= 0, bits, jnp.int32(_INT_MIN) - bits)

    def cnt_ge(t):
        return jnp.sum((key >= t).astype(jnp.int32))

    t0 = jnp.where(cnt_ge(jnp.int32(0)) >= _K, jnp.int32(0),
                   jnp.int32(_INT_MIN))

    def t_body(k, t):
        cand = t | jax.lax.shift_left(jnp.int32(1), jnp.int32(30) - k)
        return jnp.where(cnt_ge(cand) >= _K, cand, t)

    tstar = jax.lax.fori_loop(0, 31, t_body, t0, unroll=True)

    gt = key > tstar
    eq = key == tstar
    need = _K - jnp.sum(gt.astype(jnp.int32))
    idx = (jax.lax.broadcasted_iota(jnp.int32, (_F, _L), 0) * _L
           + jax.lax.broadcasted_iota(jnp.int32, (_F, _L), 1))

    def j_body(k, jm):
        cand = jm | jax.lax.shift_left(jnp.int32(1), jnp.int32(16) - k)
        c = jnp.sum((eq & (idx < cand)).astype(jnp.int32))
        return jnp.where(c < need, cand, jm)

    jm = jax.lax.fori_loop(0, 17, j_body, jnp.int32(0), unroll=True)
    mask = gt | (eq & (idx <= jm))

    masked = jnp.where(mask, score, -jnp.inf)
    m = jnp.max(masked)
    e = jnp.where(mask, jnp.exp(score - m), 0.0)
    Z = jnp.sum(e)

    As = []
    for i in range(_T):
        if i == 0:
            As.append(e)
        else:
            zeros = jnp.zeros((_F, i), jnp.float32)
            bodyp = e[:, :_L - 1 - i]
            tailp = jnp.sum(e[:, _L - 1 - i:], axis=1, keepdims=True)
            As.append(jnp.concatenate([zeros, bodyp, tailp], axis=1))
    A = jnp.stack(As, axis=0).reshape(_T, _N)

    outdot = jax.lax.dot_general(Xm, A, (((1,), (1,)), ((), ())),
                                 preferred_element_type=jnp.float32)
    G = (w / Z) * outdot + (0.5 - w) * Qt
    out_ref[0] = G.T


def kernel(x, w):
    B = x.shape[0]
    w2 = jnp.asarray(w, jnp.float32).reshape(1, 1)
    out = pl.pallas_call(
        _attn_kernel,
        grid=(B,),
        in_specs=[
            pl.BlockSpec((1, _C, _F, _L), lambda b: (b, 0, 0, 0)),
            pl.BlockSpec((1, 1), lambda b: (0, 0)),
        ],
        out_specs=pl.BlockSpec((1, _T, _C), lambda b: (b, 0, 0)),
        out_shape=jax.ShapeDtypeStruct((B, _T, _C), jnp.float32),
        compiler_params=pltpu.CompilerParams(
            dimension_semantics=("parallel",)),
    )(x, w2)
    return out.reshape(B, _C, 1, _T)
